# Initial kernel scaffold; baseline (speedup 1.0000x reference)
import jax
import jax.numpy as jnp
from jax.experimental import pallas as pl


def kernel(input_dense, input_cat, emb_tables, bot_Ws, bot_bs, top_Ws, top_bs):
    raise NotImplementedError("write your pallas kernel here")


# speedup vs baseline: 2.2180x; 2.2180x over previous
import functools

import numpy as np
import jax
import jax.numpy as jnp
from jax import lax
from jax.experimental import pallas as pl
from jax.experimental.pallas import tpu as pltpu
from jax.experimental.pallas import tpu_sc as plsc

B = 4096
F = 26
V = 100000
D = 32
NFEAT = F + 1
NPAIR = NFEAT * (NFEAT - 1) // 2

NW = 32
PER_W = (B * F) // NW
CH = 128
NCH = PER_W // CH


def _sc_gather(cat3d, table_flat):
    mesh = plsc.VectorSubcoreMesh(core_axis_name="c", subcore_axis_name="s")

    @functools.partial(
        pl.kernel,
        mesh=mesh,
        out_type=jax.ShapeDtypeStruct((B * F, D), jnp.float32),
        scratch_types=[
            pltpu.VMEM((NCH, CH), jnp.int32),
            pltpu.VMEM((PER_W, D), jnp.float32),
            pltpu.SemaphoreType.DMA,
        ],
        compiler_params=pltpu.CompilerParams(use_tc_tiling_on_sc=False),
    )
    def gather_k(cat_hbm, table_hbm, out_hbm, idx_v, rows_v, sem):
        wid = lax.axis_index("s") * 2 + lax.axis_index("c")
        base = wid * PER_W
        pltpu.sync_copy(cat_hbm.at[wid], idx_v)
        lanes = lax.iota(jnp.int32, 16)
        for c in range(NCH):
            for j in range(CH // 16):
                k0 = c * CH + j * 16
                fld = lax.rem(lanes + k0, F)
                idx_v[c, pl.ds(j * 16, 16)] = (
                    idx_v[c, pl.ds(j * 16, 16)] + fld * V
                )
        copies = [
            pltpu.async_copy(
                table_hbm.at[idx_v.at[c]],
                rows_v.at[pl.ds(c * CH, CH)],
                sem,
            )
            for c in range(NCH)
        ]
        for cp in copies:
            cp.wait()
        pltpu.sync_copy(rows_v, out_hbm.at[pl.ds(base, PER_W)])

    return gather_k(cat3d, table_flat)


BB = 256


def _mm(a, b):
    return lax.dot_general(a, b, (((1,), (0,)), ((), ())),
                           preferred_element_type=jnp.float32)


def _tc_body(dense_ref, emb_ref, bw0, bb0, bw1, bb1, bw2, bb2,
             w0d, w0s, tb0, tw1, tb1, tw2, tb2, tw3, tb3, tw4, tb4,
             out_ref):
    x = dense_ref[...]
    h = jnp.maximum(_mm(x, bw0[...]) + bb0[...], 0.0)
    h = jnp.maximum(_mm(h, bw1[...]) + bb1[...], 0.0)
    dx = _mm(h, bw2[...]) + bb2[...]

    C = jnp.concatenate([emb_ref[...], dx[:, None, :]], axis=1)
    G = lax.dot_general(C, C, (((2,), (2,)), ((0,), (0,))),
                        preferred_element_type=jnp.float32)
    Gf = G.reshape(BB, NFEAT * NFEAT)

    z = _mm(dx, w0d[...]) + _mm(Gf, w0s[...]) + tb0[...]
    h = jnp.maximum(z, 0.0)
    h = jnp.maximum(_mm(h, tw1[...]) + tb1[...], 0.0)
    h = jnp.maximum(_mm(h, tw2[...]) + tb2[...], 0.0)
    h = jnp.maximum(_mm(h, tw3[...]) + tb3[...], 0.0)
    logits = _mm(h, tw4[...]) + tb4[...]
    m = jnp.max(logits, axis=-1, keepdims=True)
    e = jnp.exp(logits - m)
    out_ref[...] = e / jnp.sum(e, axis=-1, keepdims=True)


def _tc_forward(dense, emb3, params):
    grid = B // BB

    def wspec(shape):
        return pl.BlockSpec(shape, lambda i: tuple(0 for _ in shape))

    in_specs = [
        pl.BlockSpec((BB, 13), lambda i: (i, 0)),
        pl.BlockSpec((BB, F, D), lambda i: (i, 0, 0)),
    ] + [wspec(p.shape) for p in params]

    return pl.pallas_call(
        _tc_body,
        grid=(grid,),
        in_specs=in_specs,
        out_specs=pl.BlockSpec((BB, 1), lambda i: (i, 0)),
        out_shape=jax.ShapeDtypeStruct((B, 1), jnp.float32),
    )(dense, emb3, *params)


_PAIRS = np.zeros((NFEAT, NFEAT), np.int32)
_IU = np.triu_indices(NFEAT, 1)
_PAIRS[_IU] = np.arange(1, NPAIR + 1)
_PAIRS[(_IU[1], _IU[0])] = np.arange(1, NPAIR + 1)
_PAIRS_FLAT = jnp.asarray(_PAIRS.reshape(-1))


def kernel(input_dense, input_cat, emb_tables, bot_Ws, bot_bs, top_Ws, top_bs):
    cat3d = input_cat.reshape(NW, NCH, CH)
    table_flat = emb_tables.reshape(F * V, D)
    emb3 = _sc_gather(cat3d, table_flat).reshape(B, F, D)

    w0 = top_Ws[0]
    w0d = w0[:D]
    w0pad = jnp.concatenate([jnp.zeros((1, w0.shape[1]), jnp.float32),
                             0.5 * w0[D:]], axis=0)
    w0s = w0pad[_PAIRS_FLAT]

    params = [
        bot_Ws[0], bot_bs[0][None, :],
        bot_Ws[1], bot_bs[1][None, :],
        bot_Ws[2], bot_bs[2][None, :],
        w0d, w0s, top_bs[0][None, :],
        top_Ws[1], top_bs[1][None, :],
        top_Ws[2], top_bs[2][None, :],
        top_Ws[3], top_bs[3][None, :],
        top_Ws[4], top_bs[4][None, :],
    ]
    return _tc_forward(input_dense, emb3, params)


# speedup vs baseline: 2.2654x; 1.0213x over previous
import functools

import numpy as np
import jax
import jax.numpy as jnp
from jax import lax
from jax.experimental import pallas as pl
from jax.experimental.pallas import tpu as pltpu
from jax.experimental.pallas import tpu_sc as plsc

B = 4096
F = 26
V = 100000
D = 32
NFEAT = F + 1
NPAIR = NFEAT * (NFEAT - 1) // 2

NW = 32
PER_W = (B * F) // NW
CH = 128
NCH = PER_W // CH


def _sc_gather(cat_t3, tables):
    mesh = plsc.VectorSubcoreMesh(core_axis_name="c", subcore_axis_name="s")

    @functools.partial(
        pl.kernel,
        mesh=mesh,
        out_type=jax.ShapeDtypeStruct((B, 32, 128), jnp.float32),
        scratch_types=[
            pltpu.VMEM((F, CH), jnp.int32),
            pltpu.VMEM((F, CH, D), jnp.float32),
            pltpu.SemaphoreType.DMA,
        ],
        compiler_params=pltpu.CompilerParams(use_tc_tiling_on_sc=False),
    )
    def gather_k(cat_hbm, table_hbm, out_hbm, idx_v, rows_v, sem):
        wid = lax.axis_index("s") * 2 + lax.axis_index("c")
        b0 = wid * CH
        for f in range(F):
            pltpu.sync_copy(cat_hbm.at[f].at[wid], idx_v.at[f])
        copies = [
            pltpu.async_copy(
                table_hbm.at[f].at[idx_v.at[f]],
                rows_v.at[f],
                sem,
            )
            for f in range(F)
        ]
        for cp in copies:
            cp.wait()
        for f in range(F):
            pltpu.sync_copy(
                rows_v.at[f], out_hbm.at[pl.ds(b0, CH), f, pl.ds(0, D)]
            )

    return gather_k(cat_t3, tables)


BB = 256


def _mm(a, b):
    return lax.dot_general(a, b, (((1,), (0,)), ((), ())),
                           preferred_element_type=jnp.float32)


def _tc_body(dense_ref, emb_ref, bw0, bb0, bw1, bb1, bw2, bb2,
             w0d, w0s, tb0, tw1, tb1, tw2, tb2, tw3, tb3, tw4, tb4,
             out_ref):
    x = dense_ref[...]
    h = jnp.maximum(_mm(x, bw0[...]) + bb0[...], 0.0)
    h = jnp.maximum(_mm(h, bw1[...]) + bb1[...], 0.0)
    dx = _mm(h, bw2[...]) + bb2[...]

    emb = emb_ref[...][:, :F, :D]
    C = jnp.concatenate([emb, dx[:, None, :]], axis=1)
    G = lax.dot_general(C, C, (((2,), (2,)), ((0,), (0,))),
                        preferred_element_type=jnp.float32)
    Gf = G.reshape(BB, NFEAT * NFEAT)

    z = _mm(dx, w0d[...]) + _mm(Gf, w0s[...]) + tb0[...]
    h = jnp.maximum(z, 0.0)
    h = jnp.maximum(_mm(h, tw1[...]) + tb1[...], 0.0)
    h = jnp.maximum(_mm(h, tw2[...]) + tb2[...], 0.0)
    h = jnp.maximum(_mm(h, tw3[...]) + tb3[...], 0.0)
    logits = _mm(h, tw4[...]) + tb4[...]
    m = jnp.max(logits, axis=-1, keepdims=True)
    e = jnp.exp(logits - m)
    out_ref[...] = e / jnp.sum(e, axis=-1, keepdims=True)


def _tc_forward(dense, emb3, params):
    grid = B // BB

    def wspec(shape):
        return pl.BlockSpec(shape, lambda i: tuple(0 for _ in shape))

    in_specs = [
        pl.BlockSpec((BB, 13), lambda i: (i, 0)),
        pl.BlockSpec((BB, 32, 128), lambda i: (i, 0, 0)),
    ] + [wspec(p.shape) for p in params]

    return pl.pallas_call(
        _tc_body,
        grid=(grid,),
        in_specs=in_specs,
        out_specs=pl.BlockSpec((BB, 1), lambda i: (i, 0)),
        out_shape=jax.ShapeDtypeStruct((B, 1), jnp.float32),
    )(dense, emb3, *params)


_PAIRS = np.zeros((NFEAT, NFEAT), np.int32)
_IU = np.triu_indices(NFEAT, 1)
_PAIRS[_IU] = np.arange(1, NPAIR + 1)
_PAIRS[(_IU[1], _IU[0])] = np.arange(1, NPAIR + 1)
_PAIRS_FLAT = _PAIRS.reshape(-1)


def kernel(input_dense, input_cat, emb_tables, bot_Ws, bot_bs, top_Ws, top_bs):
    cat_t3 = input_cat.T.reshape(F, NW, CH)
    emb3 = _sc_gather(cat_t3, emb_tables)

    w0 = top_Ws[0]
    w0d = w0[:D]
    w0pad = jnp.concatenate([jnp.zeros((1, w0.shape[1]), jnp.float32),
                             0.5 * w0[D:]], axis=0)
    w0s = w0pad[_PAIRS_FLAT]

    params = [
        bot_Ws[0], bot_bs[0][None, :],
        bot_Ws[1], bot_bs[1][None, :],
        bot_Ws[2], bot_bs[2][None, :],
        w0d, w0s, top_bs[0][None, :],
        top_Ws[1], top_bs[1][None, :],
        top_Ws[2], top_bs[2][None, :],
        top_Ws[3], top_bs[3][None, :],
        top_Ws[4], top_bs[4][None, :],
    ]
    return _tc_forward(input_dense, emb3, params)


# speedup vs baseline: 3.7257x; 1.6446x over previous
import functools

import numpy as np
import jax
import jax.numpy as jnp
from jax import lax
from jax.experimental import pallas as pl
from jax.experimental.pallas import tpu as pltpu
from jax.experimental.pallas import tpu_sc as plsc

B = 4096
F = 26
V = 100000
D = 32
NFEAT = F + 1
NPAIR = NFEAT * (NFEAT - 1) // 2

NW = 32
PER_W = (B * F) // NW
CH = 128
NCH = PER_W // CH


def _sc_gather(cat_t, tables_t):
    mesh = plsc.VectorSubcoreMesh(core_axis_name="c", subcore_axis_name="s")
    UNITS = (F * D) // NW

    @functools.partial(
        pl.kernel,
        mesh=mesh,
        out_type=jax.ShapeDtypeStruct((F, D, B), jnp.float32),
        scratch_types=[
            pltpu.VMEM((V,), jnp.float32),
            pltpu.VMEM((B,), jnp.int32),
            pltpu.VMEM((B,), jnp.float32),
        ],
        compiler_params=pltpu.CompilerParams(
            use_tc_tiling_on_sc=False, needs_layout_passes=False
        ),
    )
    def gather_k(cat_hbm, table_hbm, out_hbm, row_v, idx_v, out_v):
        wid = lax.axis_index("s") * 2 + lax.axis_index("c")

        def unit_body(u, _):
            r = wid * UNITS + u
            f = r // D
            d = lax.rem(r, D)
            pltpu.sync_copy(cat_hbm.at[f], idx_v)
            pltpu.sync_copy(table_hbm.at[f, d], row_v)

            def chunk_body(j, _):
                cidx = idx_v[pl.ds(j * 16, 16)]
                out_v[pl.ds(j * 16, 16)] = plsc.load_gather(row_v, [cidx])
                return 0

            lax.fori_loop(0, B // 16, chunk_body, 0)
            pltpu.sync_copy(out_v, out_hbm.at[f, d])
            return 0

        lax.fori_loop(0, UNITS, unit_body, 0)

    return gather_k(cat_t, tables_t)


BB = 256


def _mm(a, b):
    return lax.dot_general(a, b, (((1,), (0,)), ((), ())),
                           preferred_element_type=jnp.float32)


def _tc_body(dense_ref, emb_ref, bw0, bb0, bw1, bb1, bw2, bb2,
             w0d, w0s, tb0, tw1, tb1, tw2, tb2, tw3, tb3, tw4, tb4,
             out_ref):
    x = dense_ref[...]
    h = jnp.maximum(_mm(x, bw0[...]) + bb0[...], 0.0)
    h = jnp.maximum(_mm(h, bw1[...]) + bb1[...], 0.0)
    dx = _mm(h, bw2[...]) + bb2[...]

    C = jnp.concatenate([emb_ref[...], dx[:, None, :]], axis=1)
    G = lax.dot_general(C, C, (((2,), (2,)), ((0,), (0,))),
                        preferred_element_type=jnp.float32)
    Gf = G.reshape(BB, NFEAT * NFEAT)

    z = _mm(dx, w0d[...]) + _mm(Gf, w0s[...]) + tb0[...]
    h = jnp.maximum(z, 0.0)
    h = jnp.maximum(_mm(h, tw1[...]) + tb1[...], 0.0)
    h = jnp.maximum(_mm(h, tw2[...]) + tb2[...], 0.0)
    h = jnp.maximum(_mm(h, tw3[...]) + tb3[...], 0.0)
    logits = _mm(h, tw4[...]) + tb4[...]
    m = jnp.max(logits, axis=-1, keepdims=True)
    e = jnp.exp(logits - m)
    out_ref[...] = e / jnp.sum(e, axis=-1, keepdims=True)


def _tc_forward(dense, emb3, params):
    grid = B // BB

    def wspec(shape):
        return pl.BlockSpec(shape, lambda i: tuple(0 for _ in shape))

    in_specs = [
        pl.BlockSpec((BB, 13), lambda i: (i, 0)),
        pl.BlockSpec((BB, F, D), lambda i: (i, 0, 0)),
    ] + [wspec(p.shape) for p in params]

    return pl.pallas_call(
        _tc_body,
        grid=(grid,),
        in_specs=in_specs,
        out_specs=pl.BlockSpec((BB, 1), lambda i: (i, 0)),
        out_shape=jax.ShapeDtypeStruct((B, 1), jnp.float32),
    )(dense, emb3, *params)


_PAIRS = np.zeros((NFEAT, NFEAT), np.int32)
_IU = np.triu_indices(NFEAT, 1)
_PAIRS[_IU] = np.arange(1, NPAIR + 1)
_PAIRS[(_IU[1], _IU[0])] = np.arange(1, NPAIR + 1)
_PAIRS_FLAT = _PAIRS.reshape(-1)


def kernel(input_dense, input_cat, emb_tables, bot_Ws, bot_bs, top_Ws, top_bs):
    emb_t = _sc_gather(input_cat.T, emb_tables.transpose(0, 2, 1))
    emb3 = emb_t.transpose(2, 0, 1)

    w0 = top_Ws[0]
    w0d = w0[:D]
    w0pad = jnp.concatenate([jnp.zeros((1, w0.shape[1]), jnp.float32),
                             0.5 * w0[D:]], axis=0)
    w0s = w0pad[_PAIRS_FLAT]

    params = [
        bot_Ws[0], bot_bs[0][None, :],
        bot_Ws[1], bot_bs[1][None, :],
        bot_Ws[2], bot_bs[2][None, :],
        w0d, w0s, top_bs[0][None, :],
        top_Ws[1], top_bs[1][None, :],
        top_Ws[2], top_bs[2][None, :],
        top_Ws[3], top_bs[3][None, :],
        top_Ws[4], top_bs[4][None, :],
    ]
    return _tc_forward(input_dense, emb3, params)
